# Initial kernel scaffold; baseline (speedup 1.0000x reference)
import jax
import jax.numpy as jnp
from jax.experimental import pallas as pl


def kernel(cluster_ids, strand_ids, cluster_tables, strand_table, pos_table, ln_gamma, ln_beta):
    raise NotImplementedError("write your pallas kernel here")


# speedup vs baseline: 9.3023x; 9.3023x over previous
import functools

import jax
import jax.numpy as jnp
from jax import lax
from jax.experimental import pallas as pl
from jax.experimental.pallas import tpu as pltpu
from jax.experimental.pallas import tpu_sc as plsc

_NC = 2
_NS = 16
_NW = _NC * _NS
_LANES = 16


def _sc_gather(table_flat, ids_flat, Hn, V, D):
    BH = ids_flat.shape[0]
    per_w = BH // _NW
    CH = 128
    n_iter = per_w // CH

    mesh = plsc.VectorSubcoreMesh(
        core_axis_name="c", subcore_axis_name="s",
        num_cores=_NC, num_subcores=_NS,
    )

    @functools.partial(
        pl.kernel,
        out_type=jax.ShapeDtypeStruct((BH, D), jnp.float32),
        mesh=mesh,
        scratch_types=[
            pltpu.VMEM((CH,), jnp.int32),
            pltpu.VMEM((CH, D), jnp.float32),
            pltpu.SemaphoreType.DMA,
        ],
    )
    def k(ids_hbm, table_hbm, out_hbm, idx_v, rows_v, sem):
        wid = lax.axis_index("s") * _NC + lax.axis_index("c")
        base = wid * per_w

        def body(i, carry):
            start = base + i * CH
            pltpu.sync_copy(ids_hbm.at[pl.ds(start, CH)], idx_v)

            def off_body(j, carry2):
                p = lax.iota(jnp.int32, 16) + (start + j * _LANES)
                h = lax.rem(p, Hn)
                sl = pl.ds(j * _LANES, _LANES)
                idx_v[sl] = idx_v[sl] + h * V
                return carry2

            lax.fori_loop(0, CH // _LANES, off_body, 0, unroll=True)
            pltpu.async_copy(table_hbm.at[idx_v], rows_v, sem).wait()
            pltpu.sync_copy(rows_v, out_hbm.at[pl.ds(start, CH)])
            return carry

        lax.fori_loop(0, n_iter, body, 0)

    return k(ids_flat, table_flat)


def _tc_post(gathered, strand_ids3, strand_table, pos_table, ln_gamma, ln_beta):
    B, Hn, D = gathered.shape
    BB = 32
    grid = (B // BB,)

    def body(g_ref, s_ref, st_ref, pt_ref, gm_ref, bt_ref, o_ref):
        x = g_ref[...]
        s = s_ref[0, 0, :].astype(jnp.float32)
        st = st_ref[...]
        semb = st[0][None, :] + s[:, None] * (st[1] - st[0])[None, :]
        x = x + pt_ref[...][None, :, :] + semb[:, None, :]
        mean = jnp.mean(x, axis=-1, keepdims=True)
        xc = x - mean
        var = jnp.mean(xc * xc, axis=-1, keepdims=True)
        y = xc * lax.rsqrt(var + 1e-5)
        o_ref[...] = y * gm_ref[...][None, None, :] + bt_ref[...][None, None, :]

    return pl.pallas_call(
        body,
        grid=grid,
        in_specs=[
            pl.BlockSpec((BB, Hn, D), lambda i: (i, 0, 0)),
            pl.BlockSpec((1, 1, BB), lambda i: (i, 0, 0)),
            pl.BlockSpec((2, D), lambda i: (0, 0)),
            pl.BlockSpec((Hn, D), lambda i: (0, 0)),
            pl.BlockSpec((D,), lambda i: (0,)),
            pl.BlockSpec((D,), lambda i: (0,)),
        ],
        out_specs=pl.BlockSpec((BB, Hn, D), lambda i: (i, 0, 0)),
        out_shape=jax.ShapeDtypeStruct((B, Hn, D), jnp.float32),
    )(gathered, strand_ids3, strand_table, pos_table, ln_gamma, ln_beta)


def kernel(cluster_ids, strand_ids, cluster_tables, strand_table, pos_table,
           ln_gamma, ln_beta):
    B, Hn = cluster_ids.shape
    _, V, D = cluster_tables.shape
    table_flat = cluster_tables.reshape(Hn * V, D)
    ids_flat = cluster_ids.reshape(B * Hn)
    gathered = _sc_gather(table_flat, ids_flat, Hn, V, D)
    BB = 32
    strand_ids3 = strand_ids.reshape(B // BB, 1, BB)
    return _tc_post(gathered.reshape(B, Hn, D), strand_ids3, strand_table,
                    pos_table, ln_gamma, ln_beta)
